# baseline (speedup 1.0000x reference)
import jax
import jax.numpy as jnp
from jax.experimental import pallas as pl

_N_TOKENS = 32768
_K = 1024
_D = 64
_BT = 512


def _vq_body(z_ref, w_ref, idx_ref, zq_ref):
    z = z_ref[...]
    w = w_ref[...]
    zsq = jnp.sum(z * z, axis=1, keepdims=True)
    wsq = jnp.sum(w * w, axis=1)
    m = jax.lax.dot_general(z, w, (((1,), (1,)), ((), ())))
    d = zsq - 2.0 * m + wsq[None, :]
    minv = jnp.min(d, axis=1, keepdims=True)
    kiota = jax.lax.broadcasted_iota(jnp.int32, (_BT, _K), 1)
    cand = jnp.where(d == minv, kiota, _K)
    idx = jnp.min(cand, axis=1)
    idx_ref[...] = idx[:, None]
    onehot = (kiota == idx[:, None]).astype(jnp.float32)
    zq_ref[...] = jax.lax.dot_general(
        onehot, w, (((1,), (0,)), ((), ())),
        precision=jax.lax.Precision.HIGHEST)


def kernel(z, W):
    nb = _N_TOKENS // _BT
    idx2d, zq = pl.pallas_call(
        _vq_body,
        grid=(nb,),
        in_specs=[
            pl.BlockSpec((_BT, _D), lambda i: (i, 0)),
            pl.BlockSpec((_K, _D), lambda i: (0, 0)),
        ],
        out_specs=[
            pl.BlockSpec((_BT, 1), lambda i: (i, 0)),
            pl.BlockSpec((_BT, _D), lambda i: (i, 0)),
        ],
        out_shape=[
            jax.ShapeDtypeStruct((_N_TOKENS, 1), jnp.int32),
            jax.ShapeDtypeStruct((_N_TOKENS, _D), jnp.float32),
        ],
    )(z, W)
    return (zq, idx2d.reshape(_N_TOKENS))


# speedup vs baseline: 1.6745x; 1.6745x over previous
import functools

import jax
import jax.numpy as jnp
from jax import lax
from jax.experimental import pallas as pl
from jax.experimental.pallas import tpu as pltpu
from jax.experimental.pallas import tpu_sc as plsc

_N_TOKENS = 32768
_K = 1024
_D = 64
_BT = 512

_SC_INFO = plsc.get_sparse_core_info()
_NW = _SC_INFO.num_cores * _SC_INFO.num_subcores
_B_PER_W = _N_TOKENS // _NW


def _vq_body(z_ref, w_ref, idx_ref):
    z = z_ref[...]
    w = w_ref[...]
    zsq = jnp.sum(z * z, axis=1, keepdims=True)
    wsq = jnp.sum(w * w, axis=1)
    m = jax.lax.dot_general(z, w, (((1,), (1,)), ((), ())))
    d = zsq - 2.0 * m + wsq[None, :]
    minv = jnp.min(d, axis=1, keepdims=True)
    kiota = jax.lax.broadcasted_iota(jnp.int32, (_BT, _K), 1)
    cand = jnp.where(d == minv, kiota, _K)
    idx = jnp.min(cand, axis=1)
    idx_ref[...] = idx[:, None]


def _argmin_indices(z, W):
    nb = _N_TOKENS // _BT
    idx2d = pl.pallas_call(
        _vq_body,
        grid=(nb,),
        in_specs=[
            pl.BlockSpec((_BT, _D), lambda i: (i, 0)),
            pl.BlockSpec((_K, _D), lambda i: (0, 0)),
        ],
        out_specs=pl.BlockSpec((_BT, 1), lambda i: (i, 0)),
        out_shape=jax.ShapeDtypeStruct((_N_TOKENS, 1), jnp.int32),
    )(z, W)
    return idx2d.reshape(_N_TOKENS)


@functools.partial(
    pl.kernel,
    out_type=jax.ShapeDtypeStruct((_N_TOKENS, _D), jnp.float32),
    mesh=plsc.VectorSubcoreMesh(core_axis_name="c", subcore_axis_name="s"),
    scratch_types=[
        pltpu.VMEM((_B_PER_W,), jnp.int32),
        pltpu.VMEM((_B_PER_W, _D), jnp.float32),
        pltpu.SemaphoreType.DMA,
    ],
    compiler_params=pltpu.CompilerParams(use_tc_tiling_on_sc=False),
)
def _sc_gather(table_hbm, idx_hbm, out_hbm, idx_v, rows_v, sem):
    wid = lax.axis_index("s") * _SC_INFO.num_cores + lax.axis_index("c")
    base = wid * _B_PER_W
    pltpu.sync_copy(idx_hbm.at[pl.ds(base, _B_PER_W)], idx_v)
    pltpu.async_copy(table_hbm.at[idx_v], rows_v, sem).wait()
    pltpu.sync_copy(rows_v, out_hbm.at[pl.ds(base, _B_PER_W)])


def kernel(z, W):
    idx = _argmin_indices(z, W)
    zq = _sc_gather(W, idx)
    return (zq, idx)
